# baseline (speedup 1.0000x reference)
import functools

import jax
import jax.numpy as jnp
from jax import lax
from jax.experimental import pallas as pl
from jax.experimental.pallas import tpu as pltpu
from jax.experimental.pallas import tpu_sc as plsc

_ENTITY_NUM = 100000
_DIM = 64
_MARGIN = 1.0
_BATCH = 16384

_NC = 2
_NS = 16
_NW = _NC * _NS
_ROWS = _BATCH // _NW
_CH = 128
_NCHUNK = _ROWS // _CH
_NGRP = _DIM // 16


def _sc_partials(idx6, ent_emb, rel_emb):
    mesh = plsc.VectorSubcoreMesh(core_axis_name="c", subcore_axis_name="s")

    @functools.partial(
        pl.kernel,
        mesh=mesh,
        out_type=jax.ShapeDtypeStruct((2, _NW, _DIM), jnp.float32),
        scratch_types=[
            pltpu.VMEM((3, _CH), jnp.int32),
            pltpu.VMEM((_CH, _DIM), jnp.float32),
            pltpu.VMEM((_CH, _DIM), jnp.float32),
            pltpu.VMEM((_CH, _DIM), jnp.float32),
            pltpu.VMEM((_DIM,), jnp.float32),
            pltpu.SemaphoreType.DMA,
        ],
        compiler_params=pltpu.CompilerParams(use_tc_tiling_on_sc=False),
    )
    def body(idx_hbm, ent_hbm, rel_hbm, out_hbm, idx_v, hv, rv, tv, out_v, sem):
        wid = lax.axis_index("s") * _NC + lax.axis_index("c")
        base = wid * _ROWS

        for l in range(2):
            acc = tuple(jnp.zeros((16,), jnp.float32) for _ in range(_NGRP))
            for c in range(_NCHUNK):
                pltpu.sync_copy(
                    idx_hbm.at[pl.ds(3 * l, 3), pl.ds(base + c * _CH, _CH)],
                    idx_v,
                )
                g0 = pltpu.async_copy(ent_hbm.at[idx_v.at[0]], hv, sem)
                g1 = pltpu.async_copy(rel_hbm.at[idx_v.at[1]], rv, sem)
                g2 = pltpu.async_copy(ent_hbm.at[idx_v.at[2]], tv, sem)
                g0.wait()
                g1.wait()
                g2.wait()

                def row(rr, carry):
                    new = []
                    for g in range(_NGRP):
                        sl = pl.ds(g * 16, 16)
                        v = hv[rr, sl] + rv[rr, sl] - tv[rr, sl]
                        new.append(carry[g] + v * v)
                    return tuple(new)

                acc = lax.fori_loop(0, _CH, row, acc)

            for g in range(_NGRP):
                out_v[pl.ds(g * 16, 16)] = acc[g]
            pltpu.sync_copy(out_v, out_hbm.at[l, wid])

    return body(idx6, ent_emb, rel_emb)


def _finish(partials):
    def body(p_ref, o_ref):
        p = p_ref[...]
        s = jnp.sum(p, axis=1)
        d = jnp.sqrt(s)
        m = jnp.maximum(d[0:1] - d[1:2] + _MARGIN, 0.0)
        o_ref[...] = jnp.sum(m, axis=1, keepdims=True) * (1.0 / _DIM)

    return pl.pallas_call(
        body,
        out_shape=jax.ShapeDtypeStruct((1, 1), jnp.float32),
    )(partials)


@jax.jit
def kernel(current_list, corrupt_list, ent_emb, rel_emb):
    idx6 = jnp.concatenate([current_list.T, corrupt_list.T], axis=0)
    partials = _sc_partials(idx6, ent_emb, rel_emb)
    loss = _finish(partials)
    return loss[0, 0]
